# baseline (speedup 1.0000x reference)
import functools

import jax
import jax.numpy as jnp
from jax import lax
from jax.experimental import pallas as pl
from jax.experimental.pallas import tpu as pltpu
from jax.experimental.pallas import tpu_sc as plsc

N = 16384
C = 512
ROW = 2 * C
NW = 32
N_SC = 8192
ROWS_PER_W = N_SC // NW
CHUNK = 16
NCHUNK = ROWS_PER_W // CHUNK
TBL = 64 * 64
TC_BLK = 256


def _body(x_hbm, lw_hbm, rtau_hbm, out_hbm,
          lw_v, rtau_v, tab_v, red_v, xbuf0, xbuf1, out_v, sem0, sem1):
    nc = 2
    wid = lax.axis_index("s") * nc + lax.axis_index("c")
    row0 = wid * ROWS_PER_W

    pltpu.sync_copy(lw_hbm, lw_v)
    pltpu.sync_copy(rtau_hbm, rtau_v)
    rtau = rtau_v[...]

    lane = lax.iota(jnp.int32, 16)
    lane16 = lane * 16
    lane17 = lane * 17
    splats = [jnp.full((16,), k, jnp.int32) for k in range(16)]

    @pl.loop(0, TBL // 16)
    def _build(i):
        base = i * 16
        idx = base + lane
        a = idx >> 6
        b = idx & 63
        d = (a - b) & 127
        lw = plsc.load_gather(lw_v, [d])
        t = jnp.maximum(a, b).astype(jnp.float32)
        val = jnp.exp(lw - (2.0 - t) * rtau)
        for k in range(16):
            tab_v[pl.ds((base + k) * 16, 16)] = jnp.take(val, splats[k])

    def phase(ci, buf, sem):
        src = x_hbm.at[pl.ds(row0 + ci * CHUNK, CHUNK)]
        pltpu.make_async_copy(src, buf, sem).wait()

        @pl.loop(0, CHUNK)
        def _rows(r):

            @pl.loop(0, C // 16,
                     init_carry=jnp.zeros((16,), jnp.float32), unroll=8)
            def _inner(cc, acc):
                v0 = buf[r, 0, pl.ds(cc * 16, 16)]
                v1 = buf[r, 1, pl.ds(cc * 16, 16)]
                a = ((1.0 - v0) * 63.0).astype(jnp.int32)
                b = ((1.0 - v1) * 63.0).astype(jnp.int32)
                j = ((a << 10) | (b << 4)) | lane
                return acc + plsc.load_gather(tab_v, [j])

            red_v[pl.ds(r * 17, 16)] = _inner

        tot = jnp.zeros((16,), jnp.float32)
        for l in range(16):
            tot = tot + plsc.load_gather(red_v, [lane17 + l])
        out_v[pl.ds(ci * CHUNK, 16)] = tot

        @pl.when(ci + 2 < NCHUNK)
        def _():
            nsrc = x_hbm.at[pl.ds(row0 + (ci + 2) * CHUNK, CHUNK)]
            pltpu.async_copy(nsrc, buf, sem)

    pltpu.async_copy(x_hbm.at[pl.ds(row0, CHUNK)], xbuf0, sem0)
    pltpu.async_copy(x_hbm.at[pl.ds(row0 + CHUNK, CHUNK)], xbuf1, sem1)

    @pl.loop(0, NCHUNK, step=2)
    def _chunks(i):
        phase(i, xbuf0, sem0)
        phase(i + 1, xbuf1, sem1)

    pltpu.sync_copy(out_v, out_hbm.at[pl.ds(row0, ROWS_PER_W)])


def _tc_body(lw_ref, rt_ref, x_ref, o_ref):
    v = x_ref[...]
    xt = jnp.floor((1.0 - v) * 63.0)
    a = xt[:, 0, :]
    b = xt[:, 1, :]
    t = jnp.maximum(a, b)
    d = (a - b).astype(jnp.int32) & 127
    lw = jnp.take_along_axis(
        jnp.broadcast_to(lw_ref[0], (TC_BLK, 128)), d, axis=1)
    cur = jnp.exp(lw - (2.0 - t) * rt_ref[0])
    o_ref[...] = jnp.sum(cur, axis=-1, keepdims=True)


@jax.jit
def kernel(x, log_w, tau_s):
    mesh = plsc.VectorSubcoreMesh(core_axis_name="c", subcore_axis_name="s")
    run = functools.partial(
        pl.kernel,
        mesh=mesh,
        compiler_params=pltpu.CompilerParams(needs_layout_passes=False),
        out_type=jax.ShapeDtypeStruct((N_SC,), jnp.float32),
        scratch_types=[
            pltpu.VMEM((128,), jnp.float32),
            pltpu.VMEM((16,), jnp.float32),
            pltpu.VMEM((TBL * 16,), jnp.float32),
            pltpu.VMEM((CHUNK * 17,), jnp.float32),
            pltpu.VMEM((CHUNK, 2, C), jnp.float32),
            pltpu.VMEM((CHUNK, 2, C), jnp.float32),
            pltpu.VMEM((ROWS_PER_W,), jnp.float32),
            pltpu.SemaphoreType.DMA,
            pltpu.SemaphoreType.DMA,
        ],
    )(_body)
    rtau = jnp.full((16,), 1.0, jnp.float32) / tau_s
    out_sc = run(x[:N_SC], log_w, rtau)

    out_tc = pl.pallas_call(
        _tc_body,
        grid=((N - N_SC) // TC_BLK,),
        in_specs=[
            pl.BlockSpec((1, 128), lambda i: (0, 0)),
            pl.BlockSpec((16,), lambda i: (0,)),
            pl.BlockSpec((TC_BLK, 2, C), lambda i: (i, 0, 0)),
        ],
        out_specs=pl.BlockSpec((TC_BLK, 1), lambda i: (i, 0)),
        out_shape=jax.ShapeDtypeStruct((N - N_SC, 1), jnp.float32),
    )(log_w.reshape(1, 128), rtau, x[N_SC:])

    return jnp.concatenate([out_sc.reshape(N_SC, 1), out_tc], axis=0)


# speedup vs baseline: 1.6235x; 1.6235x over previous
import functools

import jax
import jax.numpy as jnp
from jax import lax
from jax.experimental import pallas as pl
from jax.experimental.pallas import tpu as pltpu
from jax.experimental.pallas import tpu_sc as plsc

N = 16384
C = 512
ROW = 2 * C
NW = 32
N_SC = 8192
ROWS_PER_W = N_SC // NW
CHUNK = 16
NCHUNK = ROWS_PER_W // CHUNK
TBL = 64 * 64
TC_BLK = 256


def _body(x_hbm, lw_hbm, rtau_hbm, out_hbm,
          lw_v, rtau_v, tab_v, red_v, xbuf0, xbuf1, out_v, sem0, sem1):
    nc = 2
    wid = lax.axis_index("s") * nc + lax.axis_index("c")
    row0 = wid * ROWS_PER_W

    pltpu.sync_copy(lw_hbm, lw_v)
    pltpu.sync_copy(rtau_hbm, rtau_v)
    rtau = rtau_v[...]

    lane = lax.iota(jnp.int32, 16)
    lane16 = lane * 16
    lane17 = lane * 17
    splats = [jnp.full((16,), k, jnp.int32) for k in range(16)]

    @pl.loop(0, TBL // 16)
    def _build(i):
        base = i * 16
        idx = base + lane
        a = idx >> 6
        b = idx & 63
        d = (a - b) & 127
        lw = plsc.load_gather(lw_v, [d])
        t = jnp.maximum(a, b).astype(jnp.float32)
        val = jnp.exp(lw - (2.0 - t) * rtau)
        for k in range(16):
            tab_v[pl.ds((base + k) * 16, 16)] = jnp.take(val, splats[k])

    def phase(ci, buf, sem):
        src = x_hbm.at[pl.ds(row0 + ci * CHUNK, CHUNK)]
        pltpu.make_async_copy(src, buf, sem).wait()

        @pl.loop(0, CHUNK)
        def _rows(r):

            @pl.loop(0, C // 16,
                     init_carry=jnp.zeros((16,), jnp.float32), unroll=8)
            def _inner(cc, acc):
                v0 = buf[r, 0, pl.ds(cc * 16, 16)]
                v1 = buf[r, 1, pl.ds(cc * 16, 16)]
                a = ((1.0 - v0) * 63.0).astype(jnp.int32)
                b = ((1.0 - v1) * 63.0).astype(jnp.int32)
                j = ((a << 10) | (b << 4)) | lane
                return acc + plsc.load_gather(tab_v, [j])

            red_v[pl.ds(r * 17, 16)] = _inner

        tot = jnp.zeros((16,), jnp.float32)
        for l in range(16):
            tot = tot + plsc.load_gather(red_v, [lane17 + l])
        out_v[pl.ds(ci * CHUNK, 16)] = tot

        @pl.when(ci + 2 < NCHUNK)
        def _():
            nsrc = x_hbm.at[pl.ds(row0 + (ci + 2) * CHUNK, CHUNK)]
            pltpu.async_copy(nsrc, buf, sem)

    pltpu.async_copy(x_hbm.at[pl.ds(row0, CHUNK)], xbuf0, sem0)
    pltpu.async_copy(x_hbm.at[pl.ds(row0 + CHUNK, CHUNK)], xbuf1, sem1)

    @pl.loop(0, NCHUNK, step=2)
    def _chunks(i):
        phase(i, xbuf0, sem0)
        phase(i + 1, xbuf1, sem1)

    pltpu.sync_copy(out_v, out_hbm.at[pl.ds(row0, ROWS_PER_W)])


def _tc_body(lw_ref, rt_ref, x_ref, o_ref):
    v = x_ref[...]
    xt = jnp.floor((1.0 - v) * 63.0)
    a = xt[:, 0, :]
    b = xt[:, 1, :]
    t = jnp.maximum(a, b)
    d = (a - b).astype(jnp.int32) & 127
    lw = jnp.take_along_axis(
        jnp.broadcast_to(lw_ref[0], (TC_BLK, 128)), d, axis=1)
    cur = jnp.exp(lw - (2.0 - t) * rt_ref[0])
    o_ref[...] = jnp.sum(cur, axis=-1, keepdims=True)


@jax.jit
def kernel(x, log_w, tau_s):
    mesh = plsc.VectorSubcoreMesh(core_axis_name="c", subcore_axis_name="s")
    run = functools.partial(
        pl.kernel,
        mesh=mesh,
        compiler_params=pltpu.CompilerParams(needs_layout_passes=False),
        out_type=jax.ShapeDtypeStruct((N_SC,), jnp.float32),
        scratch_types=[
            pltpu.VMEM((128,), jnp.float32),
            pltpu.VMEM((16,), jnp.float32),
            pltpu.VMEM((TBL * 16,), jnp.float32),
            pltpu.VMEM((CHUNK * 17,), jnp.float32),
            pltpu.VMEM((CHUNK, 2, C), jnp.float32),
            pltpu.VMEM((CHUNK, 2, C), jnp.float32),
            pltpu.VMEM((ROWS_PER_W,), jnp.float32),
            pltpu.SemaphoreType.DMA,
            pltpu.SemaphoreType.DMA,
        ],
    )(_body)
    rtau = jnp.full((16,), 1.0, jnp.float32) / tau_s
    out_sc = run(x, log_w, rtau)

    out_tc = pl.pallas_call(
        _tc_body,
        grid=((N - N_SC) // TC_BLK,),
        in_specs=[
            pl.BlockSpec((1, 128), lambda i: (0, 0)),
            pl.BlockSpec((16,), lambda i: (0,)),
            pl.BlockSpec((TC_BLK, 2, C), lambda i: (N_SC // TC_BLK + i, 0, 0)),
        ],
        out_specs=pl.BlockSpec((TC_BLK, 1), lambda i: (i, 0)),
        out_shape=jax.ShapeDtypeStruct((N - N_SC, 1), jnp.float32),
    )(log_w.reshape(1, 128), rtau, x)

    return jnp.concatenate([out_sc.reshape(N_SC, 1), out_tc], axis=0)
